# baseline (speedup 1.0000x reference)
import functools

import jax
import jax.numpy as jnp
from jax import lax
from jax.experimental import pallas as pl
from jax.experimental.pallas import tpu as pltpu, tpu_sc as plsc

B = 16384
F = 26
V = 100000

_INFO = plsc.get_sparse_core_info()
NC = _INFO.num_cores
NS = _INFO.num_subcores
NW = NC * NS
RPW = B // NW
CH = 128
NCHUNK = (F * RPW) // CH
FIRE = 8


def _sc_gather_sum(table_flat, idx_prep, bias16):
    mesh = plsc.VectorSubcoreMesh(core_axis_name="c", subcore_axis_name="s")

    @functools.partial(
        pl.kernel,
        out_type=jax.ShapeDtypeStruct((B,), jnp.float32),
        mesh=mesh,
        scratch_types=[
            pltpu.VMEM((NCHUNK, CH), jnp.int32),
            pltpu.VMEM((NCHUNK, CH), jnp.float32),
            pltpu.VMEM((16,), jnp.float32),
            pltpu.VMEM((RPW,), jnp.float32),
            pltpu.SemaphoreType.DMA,
        ],
    )
    def body(table_hbm, idx_hbm, bias_hbm, out_hbm, idx_v, buf, bias_v, out_v, sem):
        wid = lax.axis_index("s") * NC + lax.axis_index("c")
        pltpu.sync_copy(idx_hbm.at[wid], idx_v)
        pltpu.sync_copy(bias_hbm, bias_v)

        def fire_block(jb, carry):
            base = jb * FIRE
            for i in range(FIRE):
                pltpu.async_copy(table_hbm.at[idx_v.at[base + i]], buf.at[base + i], sem)
            for i in range(FIRE):
                pltpu.make_async_copy(
                    table_hbm.at[idx_v.at[base + i]], buf.at[base + i], sem
                ).wait()
            return carry

        lax.fori_loop(0, NCHUNK // FIRE, fire_block, 0)

        bvec = bias_v[...]
        for c in range(RPW // CH):
            for v in range(CH // 16):
                acc = bvec
                for f in range(F):
                    acc = acc + buf[f * (RPW // CH) + c, pl.ds(v * 16, 16)]
                out_v[pl.ds(c * CH + v * 16, 16)] = acc
        pltpu.sync_copy(out_v, out_hbm.at[pl.ds(wid * RPW, RPW)])

    return body(table_flat, idx_prep, bias16)


def kernel(x, tables, bias):
    table_flat = tables.reshape(F * V)
    idx = x.astype(jnp.int32) + (jnp.arange(F, dtype=jnp.int32) * V)[None, :]
    idx_prep = idx.reshape(NW, RPW, F).transpose(0, 2, 1).reshape(NW, NCHUNK, CH)
    bias16 = jnp.broadcast_to(bias.astype(jnp.float32), (16,))
    out = _sc_gather_sum(table_flat, idx_prep, bias16)
    return out.reshape(B, 1)
